# Initial kernel scaffold; baseline (speedup 1.0000x reference)
import jax
import jax.numpy as jnp
from jax.experimental import pallas as pl


def kernel(x):
    raise NotImplementedError("write your pallas kernel here")


# speedup vs baseline: 11.9435x; 11.9435x over previous
import functools

import jax
import jax.numpy as jnp
from jax import lax
from jax.experimental import pallas as pl
from jax.experimental.pallas import tpu as pltpu
from jax.experimental.pallas import tpu_sc as plsc

K = 1024
ROWS = 64
N = 32768
L = 16
SHIFT = 19
NBINS = 1 << (31 - SHIFT)
CAP = 4096
BIG = 1 << 30
NW = 32


def _row_pass(row_v, hist_v, cv_v, ci_v, x_hbm, out_hbm, r):
    iota = lax.iota(jnp.int32, L)
    ones = jnp.ones((L,), jnp.int32)

    pltpu.sync_copy(x_hbm.at[r], row_v)

    def clr(i, _):
        hist_v[pl.ds(i * L, L)] = jnp.zeros((L,), jnp.int32)
        return 0
    lax.fori_loop(0, NBINS // L, clr, 0, unroll=4)

    def h_body(i, _):
        v = row_v[pl.ds(i * L, L)]
        bits = plsc.bitcast(v, jnp.int32) & jnp.int32(0x7FFFFFFF)
        plsc.addupdate_scatter(hist_v, [lax.shift_right_logical(bits, SHIFT)],
                               ones)
        return 0
    lax.fori_loop(0, N // L, h_body, 0, unroll=4)

    def s_body(j, carry):
        above, b_star, cnt_ge, cnt_gt = carry
        c = NBINS // L - 1 - j
        h = hist_v[pl.ds(c * L, L)]
        suf = lax.rev(plsc.cumsum(lax.rev(h, (0,))), (0,)) + above
        ge = suf >= K
        ids = iota + c * L
        b_star = jnp.maximum(b_star, jnp.max(jnp.where(ge, ids, -1)))
        cnt_ge = jnp.minimum(cnt_ge, jnp.min(jnp.where(ge, suf, BIG)))
        cnt_gt = jnp.maximum(cnt_gt, jnp.max(jnp.where(ge, -1, suf)))
        return above + jnp.sum(h), b_star, cnt_ge, cnt_gt
    _, b_star, cnt_ge, cnt_gt = lax.fori_loop(
        0, NBINS // L, s_body,
        (jnp.int32(0), jnp.int32(-1), jnp.int32(BIG), jnp.int32(-1)))
    cnt_gt = jnp.maximum(cnt_gt, 0)
    q = K - cnt_gt

    def c_body(i, cnt):
        v = row_v[pl.ds(i * L, L)]
        bits = plsc.bitcast(v, jnp.int32) & jnp.int32(0x7FFFFFFF)
        m = lax.shift_right_logical(bits, SHIFT) == b_star
        at = jnp.minimum(cnt, CAP - L)
        plsc.store_compressed(cv_v.at[pl.ds(at, L)], bits, mask=m)
        plsc.store_compressed(ci_v.at[pl.ds(at, L)], iota + i * L, mask=m)
        return cnt + jnp.sum(m.astype(jnp.int32))
    cnt = lax.fori_loop(0, N // L, c_body, jnp.int32(0), unroll=2)
    cnt = jnp.minimum(cnt, CAP)
    nv = (cnt + L - 1) // L

    def t_bit(k, prefix):
        bit = SHIFT - 1 - k
        cand = prefix | jnp.int32(1 << bit)

        def cb(i, acc):
            cw = cv_v[pl.ds(i * L, L)]
            valid = (iota + i * L) < cnt
            return acc + jnp.sum((valid & (cw >= cand)).astype(jnp.int32))
        n_ge = lax.fori_loop(0, nv, cb, jnp.int32(0))
        return jnp.where(n_ge >= q, cand, prefix)
    T = lax.fori_loop(0, SHIFT, t_bit, lax.shift_left(b_star, SHIFT))

    def e_body(i, carry):
        n_gt, n_eq = carry
        cw = cv_v[pl.ds(i * L, L)]
        valid = (iota + i * L) < cnt
        n_gt = n_gt + jnp.sum((valid & (cw > T)).astype(jnp.int32))
        n_eq = n_eq + jnp.sum((valid & (cw == T)).astype(jnp.int32))
        return n_gt, n_eq
    n_gt, n_eq = lax.fori_loop(0, nv, e_body, (jnp.int32(0), jnp.int32(0)))
    q2 = q - n_gt

    def i_bit(k, ithr):
        bit = 14 - k
        test = ithr + jnp.int32((1 << bit) - 1)

        def cb(i, acc):
            cw = cv_v[pl.ds(i * L, L)]
            ix = ci_v[pl.ds(i * L, L)]
            valid = (iota + i * L) < cnt
            m = valid & (cw == T) & (ix <= test)
            return acc + jnp.sum(m.astype(jnp.int32))
        n_le = lax.fori_loop(0, nv, cb, jnp.int32(0))
        return jnp.where(n_le < q2, ithr | jnp.int32(1 << bit), ithr)
    ithr = lax.fori_loop(0, 15, i_bit, jnp.int32(0))
    ithr = jnp.where(n_eq > q2, ithr, BIG)

    def m_body(i, _):
        v = row_v[pl.ds(i * L, L)]
        bits = plsc.bitcast(v, jnp.int32) & jnp.int32(0x7FFFFFFF)
        keep = (bits > T) | ((bits == T) & ((iota + i * L) <= ithr))
        row_v[pl.ds(i * L, L)] = jnp.where(keep, v, 0.0)
        return 0
    lax.fori_loop(0, N // L, m_body, 0, unroll=4)

    pltpu.sync_copy(row_v, out_hbm.at[r])


@functools.partial(
    pl.kernel,
    out_type=jax.ShapeDtypeStruct((ROWS, N), jnp.float32),
    mesh=plsc.VectorSubcoreMesh(core_axis_name="c", subcore_axis_name="s"),
    compiler_params=pltpu.CompilerParams(needs_layout_passes=False),
    scratch_types=[
        pltpu.VMEM((N,), jnp.float32),
        pltpu.VMEM((NBINS,), jnp.int32),
        pltpu.VMEM((CAP,), jnp.int32),
        pltpu.VMEM((CAP,), jnp.int32),
    ],
)
def _topk_mask(x_hbm, out_hbm, row_v, hist_v, cv_v, ci_v):
    wid = lax.axis_index("s") * 2 + lax.axis_index("c")
    for r_off in (0, NW):
        _row_pass(row_v, hist_v, cv_v, ci_v, x_hbm, out_hbm, wid + r_off)


def kernel(x):
    return _topk_mask(x)


# speedup vs baseline: 13.0871x; 1.0958x over previous
import functools

import jax
import jax.numpy as jnp
from jax import lax
from jax.experimental import pallas as pl
from jax.experimental.pallas import tpu as pltpu
from jax.experimental.pallas import tpu_sc as plsc

K = 1024
ROWS = 64
N = 32768
L = 16
SHIFT = 19
NBINS = 1 << (31 - SHIFT)
CAP = 4096
BIG = 1 << 30
NW = 32


def _row_pass(row_v, hist_v, cv_v, ci_v, r):
    iota = lax.iota(jnp.int32, L)
    ones = jnp.ones((L,), jnp.int32)

    def clr(i, _):
        hist_v[pl.ds(i * L, L)] = jnp.zeros((L,), jnp.int32)
        return 0
    lax.fori_loop(0, NBINS // L, clr, 0, unroll=4)

    def h_body(i, _):
        v = row_v[pl.ds(i * L, L)]
        bits = plsc.bitcast(v, jnp.int32) & jnp.int32(0x7FFFFFFF)
        plsc.addupdate_scatter(hist_v, [lax.shift_right_logical(bits, SHIFT)],
                               ones)
        return 0
    lax.fori_loop(0, N // L, h_body, 0, unroll=4)

    def s_body(j, carry):
        above, b_star, cnt_gt = carry
        c = NBINS // L - 1 - j
        h = hist_v[pl.ds(c * L, L)]
        suf = lax.rev(plsc.cumsum(lax.rev(h, (0,))), (0,)) + above
        ge = suf >= K
        ids = iota + c * L
        b_star = jnp.maximum(b_star, jnp.max(jnp.where(ge, ids, -1)))
        cnt_gt = jnp.maximum(cnt_gt, jnp.max(jnp.where(ge, -1, suf)))
        return above + jnp.sum(h), b_star, cnt_gt
    _, b_star, cnt_gt = lax.fori_loop(
        0, NBINS // L, s_body,
        (jnp.int32(0), jnp.int32(-1), jnp.int32(-1)))
    cnt_gt = jnp.maximum(cnt_gt, 0)
    q = K - cnt_gt

    def c_body(i, cnt):
        v = row_v[pl.ds(i * L, L)]
        bits = plsc.bitcast(v, jnp.int32) & jnp.int32(0x7FFFFFFF)
        binv = lax.shift_right_logical(bits, SHIFT)
        pend = binv == b_star
        row_v[pl.ds(i * L, L)] = jnp.where(binv > b_star, v, 0.0)
        at = jnp.minimum(cnt, CAP - L)
        plsc.store_compressed(cv_v.at[pl.ds(at, L)], v, mask=pend)
        plsc.store_compressed(ci_v.at[pl.ds(at, L)], iota + i * L, mask=pend)
        return cnt + jnp.sum(pend.astype(jnp.int32))
    cnt = lax.fori_loop(0, N // L, c_body, jnp.int32(0), unroll=2)
    cnt = jnp.minimum(cnt, CAP)
    nv = (cnt + L - 1) // L

    def t_bit(k, prefix):
        bit = SHIFT - 1 - k
        cand = prefix | jnp.int32(1 << bit)

        def cb(i, acc):
            cw = plsc.bitcast(cv_v[pl.ds(i * L, L)], jnp.int32) \
                & jnp.int32(0x7FFFFFFF)
            valid = (iota + i * L) < cnt
            return acc + jnp.sum((valid & (cw >= cand)).astype(jnp.int32))
        n_ge = lax.fori_loop(0, nv, cb, jnp.int32(0))
        return jnp.where(n_ge >= q, cand, prefix)
    T = lax.fori_loop(0, SHIFT, t_bit, lax.shift_left(b_star, SHIFT))

    def e_body(i, carry):
        n_gt, n_eq = carry
        cw = plsc.bitcast(cv_v[pl.ds(i * L, L)], jnp.int32) \
            & jnp.int32(0x7FFFFFFF)
        valid = (iota + i * L) < cnt
        n_gt = n_gt + jnp.sum((valid & (cw > T)).astype(jnp.int32))
        n_eq = n_eq + jnp.sum((valid & (cw == T)).astype(jnp.int32))
        return n_gt, n_eq
    n_gt, n_eq = lax.fori_loop(0, nv, e_body, (jnp.int32(0), jnp.int32(0)))
    q2 = q - n_gt

    def i_bit(k, ithr):
        bit = 14 - k
        test = ithr + jnp.int32((1 << bit) - 1)

        def cb(i, acc):
            cw = plsc.bitcast(cv_v[pl.ds(i * L, L)], jnp.int32) \
                & jnp.int32(0x7FFFFFFF)
            ix = ci_v[pl.ds(i * L, L)]
            valid = (iota + i * L) < cnt
            m = valid & (cw == T) & (ix <= test)
            return acc + jnp.sum(m.astype(jnp.int32))
        n_le = lax.fori_loop(0, nv, cb, jnp.int32(0))
        return jnp.where(n_le < q2, ithr | jnp.int32(1 << bit), ithr)
    ithr = lax.fori_loop(0, 15, i_bit, jnp.int32(0))
    ithr = jnp.where(n_eq > q2, ithr, BIG)

    def f_body(i, _):
        cw = cv_v[pl.ds(i * L, L)]
        bits = plsc.bitcast(cw, jnp.int32) & jnp.int32(0x7FFFFFFF)
        ix = ci_v[pl.ds(i * L, L)]
        valid = (iota + i * L) < cnt
        keep = valid & ((bits > T) | ((bits == T) & (ix <= ithr)))
        plsc.store_scatter(row_v, [ix], cw, mask=keep)
        return 0
    lax.fori_loop(0, nv, f_body, 0)


@functools.partial(
    pl.kernel,
    out_type=jax.ShapeDtypeStruct((ROWS, N), jnp.float32),
    mesh=plsc.VectorSubcoreMesh(core_axis_name="c", subcore_axis_name="s"),
    compiler_params=pltpu.CompilerParams(needs_layout_passes=False),
    scratch_types=[
        pltpu.VMEM((N,), jnp.float32),
        pltpu.VMEM((N,), jnp.float32),
        pltpu.VMEM((NBINS,), jnp.int32),
        pltpu.VMEM((CAP,), jnp.float32),
        pltpu.VMEM((CAP,), jnp.int32),
        pltpu.SemaphoreType.DMA,
        pltpu.SemaphoreType.DMA,
        pltpu.SemaphoreType.DMA,
    ],
)
def _topk_mask(x_hbm, out_hbm, row0_v, row1_v, hist_v, cv_v, ci_v,
               sem0, sem1, sem2):
    wid = lax.axis_index("s") * 2 + lax.axis_index("c")
    r0 = wid
    r1 = wid + NW
    in0 = pltpu.async_copy(x_hbm.at[r0], row0_v, sem0)
    in1 = pltpu.async_copy(x_hbm.at[r1], row1_v, sem1)
    in0.wait()
    _row_pass(row0_v, hist_v, cv_v, ci_v, r0)
    out0 = pltpu.async_copy(row0_v, out_hbm.at[r0], sem2)
    in1.wait()
    _row_pass(row1_v, hist_v, cv_v, ci_v, r1)
    pltpu.sync_copy(row1_v, out_hbm.at[r1])
    out0.wait()


def kernel(x):
    return _topk_mask(x)
